# baseline (speedup 1.0000x reference)
import math

import jax
import jax.numpy as jnp
from jax.experimental import pallas as pl

_NUM_CLASSES = 3
_NUM_YAW = 2
_BOX_DOF = 7
_ROWS = 40


def _proposal_body(x_ref, w_ref, b_ref, cls_ref, reg_ref):
    rows = x_ref.shape[2]
    nx = x_ref.shape[3]
    x = x_ref[0].reshape(x_ref.shape[1], rows * nx)
    y = jax.lax.dot_general(
        w_ref[...], x,
        dimension_numbers=(((1,), (0,)), ((), ())),
        preferred_element_type=jnp.float32,
    )
    y = y + b_ref[...]
    ncy = _NUM_CLASSES * _NUM_YAW
    cls_ref[0] = y[:ncy].reshape(ncy, rows, nx)
    reg = y[ncy:].reshape(ncy, _BOX_DOF, rows, nx)
    reg = reg.transpose(0, 2, 3, 1)
    reg_ref[0] = reg.reshape(ncy, rows, nx * _BOX_DOF)


def kernel(feature_map, W_cls, b_cls, W_reg, b_reg):
    B, C, H, W = feature_map.shape
    ncy = _NUM_CLASSES * _NUM_YAW
    Wr = W_reg.reshape(_NUM_CLASSES, _BOX_DOF, _NUM_YAW, C)
    Wr = Wr.transpose(0, 2, 1, 3).reshape(ncy * _BOX_DOF, C)
    br = b_reg.reshape(_NUM_CLASSES, _BOX_DOF, _NUM_YAW)
    br = br.transpose(0, 2, 1).reshape(ncy * _BOX_DOF)
    Wall = jnp.concatenate([W_cls, Wr], axis=0)
    ball = jnp.concatenate([b_cls, br], axis=0)[:, None]

    rows = _ROWS
    grid = (B, H // rows)
    cls_out, reg_out = pl.pallas_call(
        _proposal_body,
        grid=grid,
        in_specs=[
            pl.BlockSpec((1, C, rows, W), lambda b, i: (b, 0, i, 0)),
            pl.BlockSpec((ncy * (1 + _BOX_DOF), C), lambda b, i: (0, 0)),
            pl.BlockSpec((ncy * (1 + _BOX_DOF), 1), lambda b, i: (0, 0)),
        ],
        out_specs=[
            pl.BlockSpec((1, ncy, rows, W), lambda b, i: (b, 0, i, 0)),
            pl.BlockSpec((1, ncy, rows, W * _BOX_DOF), lambda b, i: (b, 0, i, 0)),
        ],
        out_shape=[
            jax.ShapeDtypeStruct((B, ncy, H, W), jnp.float32),
            jax.ShapeDtypeStruct((B, ncy, H, W * _BOX_DOF), jnp.float32),
        ],
    )(feature_map, Wall, ball)
    cls_map = cls_out.reshape(B, _NUM_CLASSES, _NUM_YAW, H, W)
    reg_map = reg_out.reshape(B, _NUM_CLASSES, _NUM_YAW, H, W, _BOX_DOF)
    return cls_map, reg_map
